# baseline (speedup 1.0000x reference)
import functools

import jax
import jax.numpy as jnp
from jax import lax
from jax.experimental import pallas as pl
from jax.experimental.pallas import tpu as pltpu
from jax.experimental.pallas import tpu_sc as plsc

D = 1024
B, L, N_ANN = 4, 2048, 16
BL = B * L
NW = 32
TOKW = BL // NW
C = 2
NCHUNK = TOKW // C
NSLOT = 4
ROWS = 1 + N_ANN
SV = 4096 + 5
NBIN = 16
BLK = 512
GRID = BL // BLK


@functools.lru_cache(maxsize=2)
def _build_sc_gather_sum(ntok):
    tokw = ntok // NW
    nchunk = tokw // C
    mesh = plsc.VectorSubcoreMesh(core_axis_name="c", subcore_axis_name="s",
                                  num_cores=2, num_subcores=16)

    @functools.partial(
        pl.kernel,
        mesh=mesh,
        out_type=jax.ShapeDtypeStruct((ntok, D), jnp.float32),
        compiler_params=pltpu.CompilerParams(needs_layout_passes=False),
        scratch_types=[
            pltpu.VMEM((nchunk, C), jnp.int32),
            pltpu.VMEM((nchunk, C * N_ANN), jnp.int32),
            pltpu.VMEM((NSLOT, C, D), jnp.float32),
            pltpu.VMEM((NSLOT, C * N_ANN, D // 2), jnp.int32),
            pltpu.VMEM((NSLOT, C, D), jnp.float32),
        ] + [pltpu.SemaphoreType.DMA] * (3 * NSLOT),
    )
    def _sc_gather_sum(sidx_hbm, ridx_hbm, struct_w_hbm, res_p_hbm, out_hbm,
                       sidx, ridx, sbuf, rbuf, obuf, *sems):
        wid = lax.axis_index("s") * 2 + lax.axis_index("c")
        base = wid * tokw
        sem_s = sems[0:NSLOT]
        sem_r = sems[NSLOT:2 * NSLOT]
        sem_o = sems[2 * NSLOT:3 * NSLOT]
        pltpu.sync_copy(sidx_hbm.at[wid], sidx)
        pltpu.sync_copy(ridx_hbm.at[wid], ridx)

        def issue(c, slot):
            pltpu.async_copy(struct_w_hbm.at[sidx.at[c]], sbuf.at[slot],
                             sem_s[slot])
            pltpu.async_copy(res_p_hbm.at[ridx.at[c]], rbuf.at[slot],
                             sem_r[slot])

        for p in range(NSLOT - 1):
            issue(p, p)

        def body2(c2, carry):
            for b in range(NSLOT):
                c = c2 * NSLOT + b

                @pl.when(c + NSLOT - 1 < nchunk)
                def _():
                    issue(c + NSLOT - 1, (b + NSLOT - 1) % NSLOT)

                pltpu.make_async_copy(struct_w_hbm.at[sidx.at[c]],
                                      sbuf.at[b], sem_s[b]).wait()
                pltpu.make_async_copy(res_p_hbm.at[ridx.at[c]],
                                      rbuf.at[b], sem_r[b]).wait()

                @pl.when(c >= NSLOT)
                def _():
                    pltpu.make_async_copy(
                        obuf.at[b], out_hbm.at[pl.ds(base, C)], sem_o[b]).wait()

                def tree_sum(vs):
                    while len(vs) > 1:
                        nxt = [vs[i] + vs[i + 1] for i in range(0, len(vs) - 1, 2)]
                        if len(vs) % 2:
                            nxt.append(vs[-1])
                        vs = nxt
                    return vs[0]

                def vec_body(k, carry2):
                    off = k * 16
                    for t in range(C):
                        xs = [rbuf[b, t * N_ANN + j, pl.ds(off, 16)]
                              for j in range(N_ANN)]
                        lo = [plsc.bitcast(x << 16, jnp.float32) for x in xs]
                        hi = [plsc.bitcast(x, jnp.float32) for x in xs]
                        lo.append(sbuf[b, t, pl.ds(2 * off, 16)])
                        hi.append(sbuf[b, t, pl.ds(2 * off + 16, 16)])
                        obuf[b, t, pl.ds(2 * off, 16)] = tree_sum(lo)
                        obuf[b, t, pl.ds(2 * off + 16, 16)] = tree_sum(hi)
                    return carry2

                lax.fori_loop(0, D // 32, vec_body, 0, unroll=2)
                pltpu.async_copy(obuf.at[b],
                                 out_hbm.at[pl.ds(base + c * C, C)], sem_o[b])
            return carry

        lax.fori_loop(0, nchunk // NSLOT, body2, 0)
        for p in range(NSLOT):
            pltpu.make_async_copy(obuf.at[p], out_hbm.at[pl.ds(base, C)],
                                  sem_o[p]).wait()

    return _sc_gather_sum


def _tc_body(block_off, avg_ref, prp_ref, seq_ref, ss8_ref, sasa_ref,
             func_ref, res_ref, s_ref, seqw_ref, plddtw_ref, plddtb_ref,
             prpw_ref, prpb_ref, ss8w_ref, sasaw_ref, funcw_ref, resrow0_ref,
             out_ref):
    i = pl.program_id(0)
    b = (i + block_off) // (L // BLK)
    f32 = jnp.float32
    cen_col = (lax.broadcasted_iota(jnp.int32, (NBIN, 1), 0).astype(f32)
               / (NBIN - 1.0))

    avg_row = avg_ref[...]
    za = float(NBIN) * (avg_row - cen_col)
    fa = jnp.exp(-za * za)
    pe_all = lax.dot_general(fa.astype(jnp.bfloat16), plddtw_ref[...],
                             (((0,), (0,)), ((), ())),
                             preferred_element_type=f32)
    sel = (lax.broadcasted_iota(jnp.int32, (1, NBIN), 1) == b).astype(f32)
    pe = lax.dot_general(sel, pe_all, (((1,), (0,)), ((), ())),
                         preferred_element_type=f32)

    x = prp_ref[0]
    zp = float(NBIN) * (x - cen_col)
    fp = jnp.exp(-zp * zp)
    prp_e = lax.dot_general(fp.astype(jnp.bfloat16), prpw_ref[...],
                            (((0,), (0,)), ((), ())),
                            preferred_element_type=f32)

    def onehot_embed(tok_row, w):
        v = w.shape[0]
        oh = (lax.broadcasted_iota(jnp.int32, (v, 1), 0) == tok_row
              ).astype(jnp.bfloat16)
        return lax.dot_general(oh, w, (((0,), (0,)), ((), ())),
                               preferred_element_type=f32)

    seq_e = onehot_embed(seq_ref[0], seqw_ref[...])
    ss8_e = onehot_embed(ss8_ref[0], ss8w_ref[...])
    sasa_e = onehot_embed(sasa_ref[0], sasaw_ref[...])

    ft = func_ref[0]
    vf = funcw_ref.shape[1]
    parts = []
    for k in range(8):
        tr = ft[k:k + 1]
        oh = ((lax.broadcasted_iota(jnp.int32, (vf, 1), 0) == tr)
              & (tr != 0)).astype(jnp.bfloat16)
        parts.append(lax.dot_general(oh, funcw_ref[k], (((0,), (0,)), ((), ())),
                                     preferred_element_type=f32))
    func_e = jnp.concatenate(parts, axis=1)

    r = res_ref[0]
    cnt0 = jnp.sum((r == 0).astype(f32), axis=1, keepdims=True)
    corr = cnt0 * resrow0_ref[...]

    out_ref[...] = (s_ref[...] + seq_e + ss8_e + sasa_e + func_e + prp_e
                    + pe + prpb_ref[...] + plddtb_ref[...] - corr)


def _pad_rows(w, rows):
    v = w.shape[0]
    if v == rows:
        return w
    return jnp.concatenate(
        [w, jnp.zeros((rows - v,) + w.shape[1:], w.dtype)], axis=0)


def kernel(sequence_tokens, structure_tokens, average_plddt, per_res_plddt,
           ss8_tokens, sasa_tokens, function_tokens, residue_annotation_tokens,
           seq_W, plddt_W, plddt_b, prp_W, prp_b, struct_W, ss8_W, sasa_W,
           func_W, res_W):
    def pack_table(w):
        wp = w.reshape(-1, D // 32, 2, 16).transpose(0, 1, 3, 2)
        wb = wp.reshape(-1, D).astype(jnp.bfloat16)
        return lax.bitcast_convert_type(
            wb.reshape(-1, D // 2, 2), jnp.int32)

    res_p = pack_table(res_W)
    bf16 = jnp.bfloat16
    seq_wb = seq_W.astype(bf16)
    plddt_wb = plddt_W.astype(bf16)
    prp_wb = prp_W.astype(bf16)
    ss8_wp = _pad_rows(ss8_W, 16).astype(bf16)
    sasa_wp = _pad_rows(sasa_W, 24).astype(bf16)
    func_wp = jnp.concatenate(
        [func_W, jnp.zeros((8, 4, D // 8), func_W.dtype)],
        axis=1).astype(bf16)
    res_row0 = res_W[0:1, :]
    prp_b2 = prp_b.reshape(1, D)
    plddt_b2 = plddt_b.reshape(1, D)

    avg_pad = jnp.zeros((1, NBIN), jnp.float32).at[0, :B].set(average_plddt)

    def full(shape):
        return pl.BlockSpec(shape, lambda i: (0,) * len(shape))

    nhalf = BL // 2
    grid_h = nhalf // BLK
    sc_call = _build_sc_gather_sum(nhalf)

    def run_half(h):
        sl = slice(h * nhalf, (h + 1) * nhalf)
        nchunk = nhalf // NW // C
        st = structure_tokens.reshape(BL)[sl]
        rt = residue_annotation_tokens.reshape(BL, N_ANN)[sl]
        sidx = st.reshape(NW, nchunk, C).astype(jnp.int32)
        ridx = rt.reshape(NW, nchunk, C * N_ANN).astype(jnp.int32)
        s_partial = sc_call(sidx, ridx, struct_W, res_p)

        prp = per_res_plddt.reshape(BL)[sl].reshape(grid_h, 1, BLK)
        seq_t = sequence_tokens.reshape(BL)[sl].reshape(grid_h, 1, BLK).astype(jnp.int32)
        ss8_t = ss8_tokens.reshape(BL)[sl].reshape(grid_h, 1, BLK).astype(jnp.int32)
        sasa_t = sasa_tokens.reshape(BL)[sl].reshape(grid_h, 1, BLK).astype(jnp.int32)
        func_t = (function_tokens.reshape(BL, 8)[sl].T
                  .reshape(8, grid_h, BLK).transpose(1, 0, 2).astype(jnp.int32))
        res_t = rt.reshape(grid_h, BLK, N_ANN).astype(jnp.int32)

        return pl.pallas_call(
            functools.partial(_tc_body, h * grid_h),
            grid=(grid_h,),
            in_specs=[
                full((1, NBIN)),
                pl.BlockSpec((1, 1, BLK), lambda i: (i, 0, 0)),
                pl.BlockSpec((1, 1, BLK), lambda i: (i, 0, 0)),
                pl.BlockSpec((1, 1, BLK), lambda i: (i, 0, 0)),
                pl.BlockSpec((1, 1, BLK), lambda i: (i, 0, 0)),
                pl.BlockSpec((1, 8, BLK), lambda i: (i, 0, 0)),
                pl.BlockSpec((1, BLK, N_ANN), lambda i: (i, 0, 0)),
                pl.BlockSpec((BLK, D), lambda i: (i, 0)),
                full((64, D)),
                full((NBIN, D)),
                full((1, D)),
                full((NBIN, D)),
                full((1, D)),
                full((16, D)),
                full((24, D)),
                full((8, 264, D // 8)),
                full((1, D)),
            ],
            out_specs=pl.BlockSpec((BLK, D), lambda i: (i, 0)),
            out_shape=jax.ShapeDtypeStruct((nhalf, D), jnp.float32),
        )(avg_pad, prp, seq_t, ss8_t, sasa_t, func_t, res_t, s_partial,
          seq_wb, plddt_wb, plddt_b2, prp_wb, prp_b2, ss8_wp, sasa_wp,
          func_wp, res_row0)

    out = jnp.concatenate([run_half(0), run_half(1)], axis=0)
    return out.reshape(B, L, D)


# speedup vs baseline: 1.0966x; 1.0966x over previous
import functools

import jax
import jax.numpy as jnp
from jax import lax
from jax.experimental import pallas as pl
from jax.experimental.pallas import tpu as pltpu
from jax.experimental.pallas import tpu_sc as plsc

D = 1024
B, L, N_ANN = 4, 2048, 16
BL = B * L
NW = 32
TOKW = BL // NW
C = 2
NCHUNK = TOKW // C
NSLOT = 4
ROWS = 1 + N_ANN
SV = 4096 + 5
NBIN = 16
BLK = 512
GRID = BL // BLK


@functools.lru_cache(maxsize=1)
def _build_sc_gather_sum():
    mesh = plsc.VectorSubcoreMesh(core_axis_name="c", subcore_axis_name="s",
                                  num_cores=2, num_subcores=16)

    @functools.partial(
        pl.kernel,
        mesh=mesh,
        out_type=jax.ShapeDtypeStruct((BL, D), jnp.float32),
        compiler_params=pltpu.CompilerParams(needs_layout_passes=False),
        scratch_types=[
            pltpu.VMEM((NCHUNK, C), jnp.int32),
            pltpu.VMEM((NCHUNK, C * N_ANN), jnp.int32),
            pltpu.VMEM((NSLOT, C, D), jnp.float32),
            pltpu.VMEM((NSLOT, C * N_ANN, D // 2), jnp.int32),
            pltpu.VMEM((NSLOT, C, D), jnp.float32),
        ] + [pltpu.SemaphoreType.DMA] * (3 * NSLOT),
    )
    def _sc_gather_sum(sidx_hbm, ridx_hbm, struct_w_hbm, res_p_hbm, out_hbm,
                       sidx, ridx, sbuf, rbuf, obuf, *sems):
        wid = lax.axis_index("s") * 2 + lax.axis_index("c")
        base = wid * TOKW
        sem_s = sems[0:NSLOT]
        sem_r = sems[NSLOT:2 * NSLOT]
        sem_o = sems[2 * NSLOT:3 * NSLOT]
        pltpu.sync_copy(sidx_hbm.at[wid], sidx)
        pltpu.sync_copy(ridx_hbm.at[wid], ridx)

        def issue(c, slot):
            pltpu.async_copy(struct_w_hbm.at[sidx.at[c]], sbuf.at[slot],
                             sem_s[slot])
            pltpu.async_copy(res_p_hbm.at[ridx.at[c]], rbuf.at[slot],
                             sem_r[slot])

        for p in range(NSLOT - 1):
            issue(p, p)

        def body2(c2, carry):
            for b in range(NSLOT):
                c = c2 * NSLOT + b

                @pl.when(c + NSLOT - 1 < NCHUNK)
                def _():
                    issue(c + NSLOT - 1, (b + NSLOT - 1) % NSLOT)

                pltpu.make_async_copy(struct_w_hbm.at[sidx.at[c]],
                                      sbuf.at[b], sem_s[b]).wait()
                pltpu.make_async_copy(res_p_hbm.at[ridx.at[c]],
                                      rbuf.at[b], sem_r[b]).wait()

                @pl.when(c >= NSLOT)
                def _():
                    pltpu.make_async_copy(
                        obuf.at[b], out_hbm.at[pl.ds(base, C)], sem_o[b]).wait()

                def tree_sum(vs):
                    while len(vs) > 1:
                        nxt = [vs[i] + vs[i + 1] for i in range(0, len(vs) - 1, 2)]
                        if len(vs) % 2:
                            nxt.append(vs[-1])
                        vs = nxt
                    return vs[0]

                def vec_body(k, carry2):
                    off = k * 16
                    for t in range(C):
                        xs = [rbuf[b, t * N_ANN + j, pl.ds(off, 16)]
                              for j in range(N_ANN)]
                        lo = [plsc.bitcast(x << 16, jnp.float32) for x in xs]
                        hi = [plsc.bitcast(x, jnp.float32) for x in xs]
                        lo.append(sbuf[b, t, pl.ds(2 * off, 16)])
                        hi.append(sbuf[b, t, pl.ds(2 * off + 16, 16)])
                        obuf[b, t, pl.ds(2 * off, 16)] = tree_sum(lo)
                        obuf[b, t, pl.ds(2 * off + 16, 16)] = tree_sum(hi)
                    return carry2

                lax.fori_loop(0, D // 32, vec_body, 0, unroll=4)
                pltpu.async_copy(obuf.at[b],
                                 out_hbm.at[pl.ds(base + c * C, C)], sem_o[b])
            return carry

        lax.fori_loop(0, NCHUNK // NSLOT, body2, 0)
        for p in range(NSLOT):
            pltpu.make_async_copy(obuf.at[p], out_hbm.at[pl.ds(base, C)],
                                  sem_o[p]).wait()

    return _sc_gather_sum


def _tc_body(avg_ref, prp_ref, seq_ref, ss8_ref, sasa_ref, func_ref, res_ref,
             s_ref, seqw_ref, plddtw_ref, plddtb_ref, prpw_ref, prpb_ref,
             ss8w_ref, sasaw_ref, funcw_ref, resrow0_ref, out_ref):
    i = pl.program_id(0)
    b = i // (L // BLK)
    f32 = jnp.float32
    cen_col = (lax.broadcasted_iota(jnp.int32, (NBIN, 1), 0).astype(f32)
               / (NBIN - 1.0))

    avg_row = avg_ref[...]
    za = float(NBIN) * (avg_row - cen_col)
    fa = jnp.exp(-za * za)
    pe_all = lax.dot_general(fa.astype(jnp.bfloat16), plddtw_ref[...],
                             (((0,), (0,)), ((), ())),
                             preferred_element_type=f32)
    sel = (lax.broadcasted_iota(jnp.int32, (1, NBIN), 1) == b).astype(f32)
    pe = lax.dot_general(sel, pe_all, (((1,), (0,)), ((), ())),
                         preferred_element_type=f32)

    x = prp_ref[0]
    zp = float(NBIN) * (x - cen_col)
    fp = jnp.exp(-zp * zp)
    prp_e = lax.dot_general(fp.astype(jnp.bfloat16), prpw_ref[...],
                            (((0,), (0,)), ((), ())),
                            preferred_element_type=f32)

    def onehot_embed(tok_row, w):
        v = w.shape[0]
        oh = (lax.broadcasted_iota(jnp.int32, (v, 1), 0) == tok_row
              ).astype(jnp.bfloat16)
        return lax.dot_general(oh, w, (((0,), (0,)), ((), ())),
                               preferred_element_type=f32)

    seq_e = onehot_embed(seq_ref[0], seqw_ref[...])
    ss8_e = onehot_embed(ss8_ref[0], ss8w_ref[...])
    sasa_e = onehot_embed(sasa_ref[0], sasaw_ref[...])

    ft = func_ref[0]
    vf = funcw_ref.shape[1]
    parts = []
    for k in range(8):
        tr = ft[k:k + 1]
        oh = ((lax.broadcasted_iota(jnp.int32, (vf, 1), 0) == tr)
              & (tr != 0)).astype(jnp.bfloat16)
        parts.append(lax.dot_general(oh, funcw_ref[k], (((0,), (0,)), ((), ())),
                                     preferred_element_type=f32))
    func_e = jnp.concatenate(parts, axis=1)

    r = res_ref[0]
    cnt0 = jnp.sum((r == 0).astype(f32), axis=1, keepdims=True)
    corr = cnt0 * resrow0_ref[...]

    out_ref[...] = (s_ref[...] + seq_e + ss8_e + sasa_e + func_e + prp_e
                    + pe + prpb_ref[...] + plddtb_ref[...] - corr)


def _pad_rows(w, rows):
    v = w.shape[0]
    if v == rows:
        return w
    return jnp.concatenate(
        [w, jnp.zeros((rows - v,) + w.shape[1:], w.dtype)], axis=0)


def kernel(sequence_tokens, structure_tokens, average_plddt, per_res_plddt,
           ss8_tokens, sasa_tokens, function_tokens, residue_annotation_tokens,
           seq_W, plddt_W, plddt_b, prp_W, prp_b, struct_W, ss8_W, sasa_W,
           func_W, res_W):
    def pack_table(w):
        wp = w.reshape(-1, D // 32, 2, 16).transpose(0, 1, 3, 2)
        wb = wp.reshape(-1, D).astype(jnp.bfloat16)
        return lax.bitcast_convert_type(
            wb.reshape(-1, D // 2, 2), jnp.int32)

    res_p = pack_table(res_W)
    sidx = structure_tokens.reshape(NW, NCHUNK, C).astype(jnp.int32)
    ridx = residue_annotation_tokens.reshape(NW, NCHUNK, C * N_ANN).astype(jnp.int32)
    s_partial = _build_sc_gather_sum()(sidx, ridx, struct_W, res_p)

    avg_pad = jnp.zeros((1, NBIN), jnp.float32).at[0, :B].set(average_plddt)
    prp = per_res_plddt.reshape(GRID, 1, BLK)
    seq_t = sequence_tokens.reshape(GRID, 1, BLK).astype(jnp.int32)
    ss8_t = ss8_tokens.reshape(GRID, 1, BLK).astype(jnp.int32)
    sasa_t = sasa_tokens.reshape(GRID, 1, BLK).astype(jnp.int32)
    func_t = (function_tokens.reshape(BL, 8).T
              .reshape(8, GRID, BLK).transpose(1, 0, 2).astype(jnp.int32))
    res_t = residue_annotation_tokens.reshape(GRID, BLK, N_ANN).astype(jnp.int32)

    bf16 = jnp.bfloat16
    seq_wb = seq_W.astype(bf16)
    plddt_wb = plddt_W.astype(bf16)
    prp_wb = prp_W.astype(bf16)
    ss8_wp = _pad_rows(ss8_W, 16).astype(bf16)
    sasa_wp = _pad_rows(sasa_W, 24).astype(bf16)
    func_wp = jnp.concatenate(
        [func_W, jnp.zeros((8, 4, D // 8), func_W.dtype)],
        axis=1).astype(bf16)
    res_row0 = res_W[0:1, :]
    prp_b2 = prp_b.reshape(1, D)
    plddt_b2 = plddt_b.reshape(1, D)

    def full(shape):
        return pl.BlockSpec(shape, lambda i: (0,) * len(shape))

    out = pl.pallas_call(
        _tc_body,
        grid=(GRID,),
        in_specs=[
            full((1, NBIN)),
            pl.BlockSpec((1, 1, BLK), lambda i: (i, 0, 0)),
            pl.BlockSpec((1, 1, BLK), lambda i: (i, 0, 0)),
            pl.BlockSpec((1, 1, BLK), lambda i: (i, 0, 0)),
            pl.BlockSpec((1, 1, BLK), lambda i: (i, 0, 0)),
            pl.BlockSpec((1, 8, BLK), lambda i: (i, 0, 0)),
            pl.BlockSpec((1, BLK, N_ANN), lambda i: (i, 0, 0)),
            pl.BlockSpec((BLK, D), lambda i: (i, 0)),
            full((64, D)),
            full((NBIN, D)),
            full((1, D)),
            full((NBIN, D)),
            full((1, D)),
            full((16, D)),
            full((24, D)),
            full((8, 264, D // 8)),
            full((1, D)),
        ],
        out_specs=pl.BlockSpec((BLK, D), lambda i: (i, 0)),
        out_shape=jax.ShapeDtypeStruct((BL, D), jnp.float32),
    )(avg_pad, prp, seq_t, ss8_t, sasa_t, func_t, res_t, s_partial,
      seq_wb, plddt_wb, plddt_b2, prp_wb, prp_b2, ss8_wp, sasa_wp, func_wp,
      res_row0)

    return out.reshape(B, L, D)
